# baseline (speedup 1.0000x reference)
import functools

import jax
import jax.numpy as jnp
from jax import lax
from jax.experimental import pallas as pl
from jax.experimental.pallas import tpu as pltpu
from jax.experimental.pallas import tpu_sc as plsc

N_NODE = 20000
EMB = 128
B = 1024
L = 50
E = 320000
T2 = 10.0
W_K = 10.0

NC = 2
NS = 16
HALF = 64


EDGE_CHUNK = 80
EDGES_PER_SUB = E // NS
SLAB = 1000
NSLAB = N_NODE // SLAB
ZCHUNK = 200


def _spmm_body(graph_ids, x_of_k, nchunks, xs, erows, ecols, evals, out_hbm,
               acc_shared, idx_v, row_v, val_v, gbuf, zbuf, sem):
    cid = lax.axis_index("c")
    sid = lax.axis_index("s")

    def zrow(i, c):
        for j in range(HALF // 16):
            zbuf[i, pl.ds(j * 16, 16)] = jnp.zeros((16,), jnp.float32)
        return c

    lax.fori_loop(0, ZCHUNK, zrow, 0)

    def do_chunk(k, g):
        for rep in range(NSLAB // NS + 1):
            slab = rep * NS + sid

            @pl.when(slab < NSLAB)
            def _():
                for z in range(SLAB // ZCHUNK):
                    pltpu.sync_copy(
                        zbuf,
                        acc_shared.at[pl.ds(slab * SLAB + z * ZCHUNK, ZCHUNK)])
        plsc.subcore_barrier()

        ebase = sid * EDGES_PER_SUB

        def step(i, carry):
            off = ebase + i * EDGE_CHUNK
            pltpu.sync_copy(ecols[g].at[pl.ds(off, EDGE_CHUNK)], idx_v)
            pltpu.sync_copy(erows[g].at[pl.ds(off, EDGE_CHUNK)], row_v)
            pltpu.sync_copy(evals[g].at[pl.ds(off, EDGE_CHUNK)], val_v)
            pltpu.async_copy(xs[x_of_k[k]].at[idx_v], gbuf, sem).wait()

            def scale(grp, c2):
                vv = val_v[pl.ds(grp * 16, 16)]
                for l in range(16):
                    s = vv[l]
                    e = grp * 16 + l
                    for j in range(HALF // 16):
                        sl = pl.ds(j * 16, 16)
                        gbuf[e, sl] = gbuf[e, sl] * s
                return c2

            lax.fori_loop(0, EDGE_CHUNK // 16, scale, 0)
            pltpu.sync_copy(gbuf, acc_shared.at[row_v], add=True)
            return carry

        lax.fori_loop(0, EDGES_PER_SUB // EDGE_CHUNK, step, 0)
        plsc.subcore_barrier()
        for rep in range(NSLAB // NS + 1):
            slab = rep * NS + sid

            @pl.when(slab < NSLAB)
            def _():
                pltpu.sync_copy(acc_shared.at[pl.ds(slab * SLAB, SLAB)],
                                out_hbm.at[k, pl.ds(slab * SLAB, SLAB)])
        plsc.subcore_barrier()

    for k in range(nchunks):

        @pl.when(cid == (k % NC))
        def _():
            do_chunk(k, graph_ids[k])


def _spmm_sc(graph_ids, x_of_k, x_chunk_list, rows_list, cols_list,
             vals_list):
    nchunks = len(graph_ids)
    nx = len(x_chunk_list)
    mesh = plsc.VectorSubcoreMesh(core_axis_name="c", subcore_axis_name="s")
    ng = len(rows_list)

    def body(*refs):
        xs = refs[0:nx]
        erows = refs[nx:nx + ng]
        ecols = refs[nx + ng:nx + 2 * ng]
        evals = refs[nx + 2 * ng:nx + 3 * ng]
        out_hbm = refs[nx + 3 * ng]
        (acc_shared, idx_v, row_v, val_v, gbuf, zbuf, sem) = \
            refs[nx + 3 * ng + 1:]
        _spmm_body(graph_ids, x_of_k, nchunks, xs, erows, ecols, evals,
                   out_hbm, acc_shared, idx_v, row_v, val_v, gbuf, zbuf, sem)

    kern = pl.kernel(
        body,
        mesh=mesh,
        compiler_params=pltpu.CompilerParams(use_tc_tiling_on_sc=False),
        out_type=jax.ShapeDtypeStruct((nchunks, N_NODE, HALF), jnp.float32),
        scratch_types=[
            pltpu.VMEM_SHARED((N_NODE, HALF), jnp.float32),
            pltpu.VMEM((EDGE_CHUNK,), jnp.int32),
            pltpu.VMEM((EDGE_CHUNK,), jnp.int32),
            pltpu.VMEM((EDGE_CHUNK,), jnp.float32),
            pltpu.VMEM((EDGE_CHUNK, HALF), jnp.float32),
            pltpu.VMEM((ZCHUNK, HALF), jnp.float32),
            pltpu.SemaphoreType.DMA,
        ],
    )
    return kern(*x_chunk_list, *rows_list, *cols_list, *vals_list)


GIDX_CHUNK = 80


def _gather_sc(table, idx):
    n = idx.shape[0]
    per_w = n // (NC * NS)
    mesh = plsc.VectorSubcoreMesh(core_axis_name="c", subcore_axis_name="s")

    def body(table_hbm, idx_hbm, out_hbm, idx_v, rows_v, sem):
        wid = lax.axis_index("s") * NC + lax.axis_index("c")
        base = wid * per_w

        def step(i, c):
            off = base + i * GIDX_CHUNK
            pltpu.sync_copy(idx_hbm.at[pl.ds(off, GIDX_CHUNK)], idx_v)
            pltpu.async_copy(table_hbm.at[idx_v], rows_v, sem).wait()
            pltpu.sync_copy(rows_v, out_hbm.at[pl.ds(off, GIDX_CHUNK)])
            return c

        lax.fori_loop(0, per_w // GIDX_CHUNK, step, 0)

    kern = pl.kernel(
        body,
        mesh=mesh,
        out_type=jax.ShapeDtypeStruct((n, EMB), jnp.float32),
        scratch_types=[
            pltpu.VMEM((GIDX_CHUNK,), jnp.int32),
            pltpu.VMEM((GIDX_CHUNK, EMB), jnp.float32),
            pltpu.SemaphoreType.DMA,
        ],
    )
    return kern(table, idx)


def _mm_kernel(x_ref, w_ref, o_ref):
    o_ref[...] = jax.lax.dot_general(
        x_ref[...], w_ref[...], (((1,), (0,)), ((), ())),
        preferred_element_type=jnp.float32)


def _matmul(x, w, block_m=2048):
    m, k = x.shape
    n = w.shape[1]
    nblk = pl.cdiv(m, block_m)
    return pl.pallas_call(
        _mm_kernel,
        grid=(nblk,),
        in_specs=[pl.BlockSpec((block_m, k), lambda i: (i, 0)),
                  pl.BlockSpec((k, n), lambda i: (0, 0))],
        out_specs=pl.BlockSpec((block_m, n), lambda i: (i, 0)),
        out_shape=jax.ShapeDtypeStruct((m, n), jnp.float32),
    )(x, w)


def _l2n(x, eps=1e-12):
    n = jnp.sqrt(jnp.sum(x * x, axis=-1, keepdims=True))
    return x / jnp.maximum(n, eps)


def _combine_kernel(x0_ref, z1_ref, z2_ref, mixin_ref, itemn_ref):
    x0 = x0_ref[...]
    z1 = z1_ref[...]
    z2 = z2_ref[...]
    outs = []
    for b in range(5):
        sl = slice(b * EMB, (b + 1) * EMB)
        outs.append((x0[:, sl] + _l2n(z1[:, sl]) + _l2n(z2[:, sl])) / 3.0)
    item_emb = outs[0]
    image_emb = outs[1] + outs[3]
    text_emb = outs[2] + outs[4]
    mixin_ref[...] = jnp.concatenate([item_emb, image_emb, text_emb], axis=-1)
    itemn_ref[...] = _l2n(item_emb)


def _combine(x0cat, z1cat, z2cat, block_m=2000):
    nblk = N_NODE // block_m
    return pl.pallas_call(
        _combine_kernel,
        grid=(nblk,),
        in_specs=[pl.BlockSpec((block_m, 5 * EMB), lambda i: (i, 0))] * 3,
        out_specs=[pl.BlockSpec((block_m, 3 * EMB), lambda i: (i, 0)),
                   pl.BlockSpec((block_m, EMB), lambda i: (i, 0))],
        out_shape=[jax.ShapeDtypeStruct((N_NODE, 3 * EMB), jnp.float32),
                   jax.ShapeDtypeStruct((N_NODE, EMB), jnp.float32)],
    )(x0cat, z1cat, z2cat)


def _mlp_kernel(x_ref, w1_ref, b1_ref, w2_ref, b2_ref, o_ref):
    h = jnp.tanh(jax.lax.dot_general(
        x_ref[...], w1_ref[...], (((1,), (0,)), ((), ())),
        preferred_element_type=jnp.float32) + b1_ref[...])
    o_ref[...] = jnp.tanh(jax.lax.dot_general(
        h, w2_ref[...], (((1,), (0,)), ((), ())),
        preferred_element_type=jnp.float32) + b2_ref[...])


def _mlp(x, w1t, b1, w2t, b2, block_m=2000):
    nblk = N_NODE // block_m
    return pl.pallas_call(
        _mlp_kernel,
        grid=(nblk,),
        in_specs=[pl.BlockSpec((block_m, 3 * EMB), lambda i: (i, 0)),
                  pl.BlockSpec((3 * EMB, EMB), lambda i: (0, 0)),
                  pl.BlockSpec((1, EMB), lambda i: (0, 0)),
                  pl.BlockSpec((EMB, EMB), lambda i: (0, 0)),
                  pl.BlockSpec((1, EMB), lambda i: (0, 0))],
        out_specs=pl.BlockSpec((block_m, EMB), lambda i: (i, 0)),
        out_shape=jax.ShapeDtypeStruct((N_NODE, EMB), jnp.float32),
    )(x, w1t, b1.reshape(1, EMB), w2t, b2.reshape(1, EMB))


SESS_BLK = 128


def _attn_kernel(seqh_ref, len_ref, mask_ref, g1w_ref, g1b_ref, g2w_ref,
                 w2_ref, sess_ref):
    seq_h = seqh_ref[...].reshape(SESS_BLK, L, EMB)
    lens = len_ref[...].astype(jnp.float32)
    maskf = mask_ref[...][..., None]
    hs = jnp.sum(seq_h, axis=1) / lens
    flat = seq_h.reshape(SESS_BLK * L, EMB)
    nh_lin = jax.lax.dot_general(jnp.tanh(flat), g1w_ref[...],
                                 (((1,), (0,)), ((), ())),
                                 preferred_element_type=jnp.float32)
    hs_lin = jax.lax.dot_general(hs, g2w_ref[...], (((1,), (0,)), ((), ())),
                                 preferred_element_type=jnp.float32)
    nh = jax.nn.sigmoid(nh_lin.reshape(SESS_BLK, L, EMB) + g1b_ref[...]
                        + hs_lin[:, None, :])
    beta = jax.lax.dot_general(nh.reshape(SESS_BLK * L, EMB), w2_ref[...],
                               (((1,), (0,)), ((), ())),
                               preferred_element_type=jnp.float32)
    beta = beta.reshape(SESS_BLK, L, 1) * maskf
    select = jnp.sum(beta * seq_h, axis=1)

    pos = lax.broadcasted_iota(jnp.int32, (SESS_BLK, L), 1).astype(jnp.float32)
    lens_b = lens
    order = jnp.where(pos < lens_b, lens_b - pos, 0.0)
    new_order = jnp.exp(order / T2)
    last = seq_h[:, 0:1, :]
    dot = jnp.sum(seq_h * last, axis=-1)
    na = jnp.sqrt(jnp.sum(seq_h * seq_h, axis=-1))
    nb = jnp.sqrt(jnp.sum(last * last, axis=-1))
    cs = dot / (jnp.maximum(na, 1e-8) * jnp.maximum(nb, 1e-8))
    weights = new_order * cs
    wmask = jnp.where(weights != 0, weights, -9e10)
    wmax = jnp.max(wmask, axis=1, keepdims=True)
    ew = jnp.exp(wmask - wmax)
    fw = ew / jnp.sum(ew, axis=1, keepdims=True)
    session_aw = jnp.sum(fw[..., None] * seq_h, axis=1)
    sess_ref[...] = select + session_aw


def _attention(seq_h_flat, session_len, mask, g1w_t, g1b, g2w_t, w_2):
    nblk = B // SESS_BLK
    return pl.pallas_call(
        _attn_kernel,
        grid=(nblk,),
        in_specs=[pl.BlockSpec((SESS_BLK * L, EMB), lambda i: (i, 0)),
                  pl.BlockSpec((SESS_BLK, 1), lambda i: (i, 0)),
                  pl.BlockSpec((SESS_BLK, L), lambda i: (i, 0)),
                  pl.BlockSpec((EMB, EMB), lambda i: (0, 0)),
                  pl.BlockSpec((1, EMB), lambda i: (0, 0)),
                  pl.BlockSpec((EMB, EMB), lambda i: (0, 0)),
                  pl.BlockSpec((EMB, 1), lambda i: (0, 0))],
        out_specs=pl.BlockSpec((SESS_BLK, EMB), lambda i: (i, 0)),
        out_shape=jax.ShapeDtypeStruct((B, EMB), jnp.float32),
    )(seq_h_flat, session_len, mask, g1w_t, g1b.reshape(1, EMB), g2w_t, w_2)


def _simtop_kernel(sess_ref, out_ref):
    s = sess_ref[...]
    fenzi = jax.lax.dot_general(s, s, (((1,), (1,)), ((), ())),
                                preferred_element_type=jnp.float32)
    fenmu_l = jnp.sqrt(jnp.sum(s * s + 1e-6, axis=1, keepdims=True))
    denom = fenmu_l * fenmu_l.reshape(1, B)
    logits = fenzi / denom
    lmax = jnp.max(logits, axis=1, keepdims=True)
    el = jnp.exp(logits - lmax)
    cos_sim = el / jnp.sum(el, axis=1, keepdims=True)

    cols = lax.broadcasted_iota(jnp.int32, (B, B), 1)
    work = cos_sim
    vals = []
    hots = []
    for _ in range(3):
        v = jnp.max(work, axis=1, keepdims=True)
        ismax = work == v
        first = jnp.min(jnp.where(ismax, cols, B), axis=1, keepdims=True)
        hot = (cols == first).astype(jnp.float32)
        vals.append(v)
        hots.append(hot)
        work = jnp.where(hot > 0, -jnp.inf, work)
    v3 = jnp.concatenate(vals, axis=1)
    vmax = jnp.max(v3, axis=1, keepdims=True)
    ev = jnp.exp(v3 - vmax)
    wsm = ev / jnp.sum(ev, axis=1, keepdims=True)
    m = (wsm[:, 0:1] * hots[0] + wsm[:, 1:2] * hots[1]
         + wsm[:, 2:3] * hots[2])
    neighbor = jax.lax.dot_general(m, s, (((1,), (0,)), ((), ())),
                                   preferred_element_type=jnp.float32)
    sess_final = s + _l2n(neighbor)
    out_ref[...] = W_K * _l2n(sess_final)


def _simtop(sess_emb):
    return pl.pallas_call(
        _simtop_kernel,
        out_shape=jax.ShapeDtypeStruct((B, EMB), jnp.float32),
    )(sess_emb)


def _scores_kernel(sess_ref, itemn_ref, o_ref):
    o_ref[...] = jax.lax.dot_general(
        sess_ref[...], itemn_ref[...], (((1,), (1,)), ((), ())),
        preferred_element_type=jnp.float32)


def _scores(sess, item_n, block_n=4096):
    nblk = pl.cdiv(N_NODE, block_n)
    return pl.pallas_call(
        _scores_kernel,
        grid=(nblk,),
        in_specs=[pl.BlockSpec((B, EMB), lambda i: (0, 0)),
                  pl.BlockSpec((block_n, EMB), lambda i: (i, 0))],
        out_specs=pl.BlockSpec((B, block_n), lambda i: (0, i)),
        out_shape=jax.ShapeDtypeStruct((B, N_NODE), jnp.float32),
    )(sess, item_n)


def kernel(session_item, session_len, reversed_sess_item, mask, embedding,
           image_pca, text_pca, adj_rows, adj_cols, adj_vals, img_rows,
           img_cols, img_vals, txt_rows, txt_cols, txt_vals, W_ic0, W_ic1,
           glu1_w, glu1_b, glu2_w, w_2, mlp1_w, mlp1_b, mlp2_w, mlp2_b):
    rows_list = [adj_rows, img_rows, txt_rows]
    cols_list = [adj_cols, img_cols, txt_cols]
    vals_list = [adj_vals, img_vals, txt_vals]

    x3 = jnp.concatenate([embedding, image_pca, text_pca], axis=0)
    y1 = _matmul(x3, W_ic0.T)
    y1c = y1.reshape(3, N_NODE, 2, HALF).transpose(0, 2, 1, 3)
    y1c = y1c.reshape(6, N_NODE, HALF)
    x_idx1 = [0, 1, 2, 3, 4, 5, 2, 3, 4, 5]
    graphs = [0, 0, 0, 0, 0, 0, 1, 1, 2, 2]
    z1 = _spmm_sc(graphs, x_idx1, [y1c[i] for i in range(6)], rows_list,
                  cols_list, vals_list)
    z1b = z1.reshape(5, 2, N_NODE, HALF).transpose(0, 2, 1, 3)
    z1b = z1b.reshape(5, N_NODE, EMB)

    y2 = _matmul(z1b.reshape(5 * N_NODE, EMB), W_ic1.T)
    y2c = y2.reshape(5, N_NODE, 2, HALF).transpose(0, 2, 1, 3)
    y2c = y2c.reshape(10, N_NODE, HALF)
    z2 = _spmm_sc(graphs, list(range(10)), [y2c[i] for i in range(10)],
                  rows_list, cols_list, vals_list)
    z2b = z2.reshape(5, 2, N_NODE, HALF).transpose(0, 2, 1, 3)
    z2b = z2b.reshape(5, N_NODE, EMB)

    x0cat = jnp.concatenate([embedding, image_pca, text_pca, image_pca,
                             text_pca], axis=1)
    z1cat = jnp.concatenate([z1b[i] for i in range(5)], axis=1)
    z2cat = jnp.concatenate([z2b[i] for i in range(5)], axis=1)
    mixin, item_n = _combine(x0cat, z1cat, z2cat)

    mix = _mlp(mixin, mlp1_w.T, mlp1_b, mlp2_w.T, mlp2_b)

    table = jnp.concatenate([jnp.zeros((1, EMB), jnp.float32), mix], axis=0)
    seq_h_flat = _gather_sc(table, reversed_sess_item.reshape(B * L)
                            .astype(jnp.int32))
    sess_emb = _attention(seq_h_flat, session_len, mask, glu1_w.T, glu1_b,
                          glu2_w.T, w_2)
    sess = _simtop(sess_emb)
    return _scores(sess, item_n)


# speedup vs baseline: 1.8517x; 1.8517x over previous
import functools

import jax
import jax.numpy as jnp
from jax import lax
from jax.experimental import pallas as pl
from jax.experimental.pallas import tpu as pltpu
from jax.experimental.pallas import tpu_sc as plsc

N_NODE = 20000
EMB = 128
B = 1024
L = 50
E = 320000
T2 = 10.0
W_K = 10.0

NC = 2
NS = 16
HALF = 64


EDGE_CHUNK = 80
EDGES_PER_SUB = E // NS
SLAB = 1000
NSLAB = N_NODE // SLAB
ZCHUNK = 40
ESTAGE = 10000


def _spmm_body(nchunks, xref, ecols, erows, evals, out_hbm,
               acc_shared, idx_v, row_v, val_v, gbuf, gbuf2, zbuf,
               sem, sem2, sem3, sem4):
    cid = lax.axis_index("c")
    sid = lax.axis_index("s")

    def zrow(i, c):
        for j in range(HALF // 16):
            zbuf[i, pl.ds(j * 16, 16)] = jnp.zeros((16,), jnp.float32)
        return c

    lax.fori_loop(0, ZCHUNK, zrow, 0)

    def do_chunk(i_chunk, carry):
        k = i_chunk * NC + cid
        for rep in range(NSLAB // NS + 1):
            slab = rep * NS + sid

            @pl.when(slab < NSLAB)
            def _():
                def zs(z, c):
                    pltpu.sync_copy(
                        zbuf,
                        acc_shared.at[pl.ds(slab * SLAB + z * ZCHUNK, ZCHUNK)])
                    return c

                lax.fori_loop(0, SLAB // ZCHUNK, zs, 0)
        plsc.subcore_barrier()

        gb = (gbuf, gbuf2)
        gsem = (sem, sem2)
        ssem = (sem3, sem4)
        nstep = ESTAGE // EDGE_CHUNK

        def gather(i, p):
            pltpu.async_copy(
                xref.at[idx_v.at[pl.ds(i * EDGE_CHUNK, EDGE_CHUNK)]],
                gb[p], gsem[p])

        def scale(i, p):
            buf = gb[p]

            def grp_body(grp, c2):
                vv = val_v[pl.ds(i * EDGE_CHUNK + grp * 16, 16)]
                for l in range(16):
                    s = vv[l]
                    e = grp * 16 + l
                    for j in range(HALF // 16):
                        sl = pl.ds(j * 16, 16)
                        buf[e, sl] = buf[e, sl] * s
                return c2

            lax.fori_loop(0, EDGE_CHUNK // 16, grp_body, 0)

        def scatter(i, p):
            for q in range(EDGE_CHUNK // 16):
                idx16 = row_v[pl.ds(i * EDGE_CHUNK + q * 16, 16)]
                pltpu.async_copy(gb[p].at[pl.ds(q * 16, 16)],
                                 acc_shared.at[idx16], ssem[p], add=True)

        def gwait(i, p):
            pltpu.make_async_copy(
                xref.at[idx_v.at[pl.ds(i * EDGE_CHUNK, EDGE_CHUNK)]],
                gb[p], gsem[p]).wait()

        def swait(i, p):
            for q in range(EDGE_CHUNK // 16):
                idx16 = row_v[pl.ds(i * EDGE_CHUNK + q * 16, 16)]
                pltpu.make_async_copy(gb[p].at[pl.ds(q * 16, 16)],
                                      acc_shared.at[idx16], ssem[p]).wait()

        def pair(j, carry):
            e = 2 * j
            gwait(e, 0)
            gather(e + 1, 1)
            scale(e, 0)
            scatter(e, 0)
            swait(e, 0)
            gwait(e + 1, 1)
            gather(e + 2, 0)
            scale(e + 1, 1)
            scatter(e + 1, 1)
            swait(e + 1, 1)
            return carry

        def half(h, carry):
            ebase = sid * EDGES_PER_SUB + h * ESTAGE
            pltpu.sync_copy(ecols.at[k, pl.ds(ebase, ESTAGE)], idx_v)
            pltpu.sync_copy(erows.at[k, pl.ds(ebase, ESTAGE)], row_v)
            pltpu.sync_copy(evals.at[k, pl.ds(ebase, ESTAGE)], val_v)
            gather(0, 0)
            lax.fori_loop(0, (nstep - 1) // 2, pair, 0)
            e = nstep - 1
            gwait(e, 0)
            scale(e, 0)
            scatter(e, 0)
            swait(e, 0)
            return carry

        lax.fori_loop(0, EDGES_PER_SUB // ESTAGE, half, 0)
        plsc.subcore_barrier()
        for rep in range(NSLAB // NS + 1):
            slab = rep * NS + sid

            @pl.when(slab < NSLAB)
            def _():
                pltpu.sync_copy(acc_shared.at[pl.ds(slab * SLAB, SLAB)],
                                out_hbm.at[k, pl.ds(slab * SLAB, SLAB)])
        plsc.subcore_barrier()
        return carry

    lax.fori_loop(0, nchunks // NC, do_chunk, 0)


def _spmm_sc(xflat, cols_k, rows_k, vals_k):
    nchunks = cols_k.shape[0]
    mesh = plsc.VectorSubcoreMesh(core_axis_name="c", subcore_axis_name="s")

    def body(xref, ecols, erows, evals, out_hbm, *scratch):
        _spmm_body(nchunks, xref, ecols, erows, evals, out_hbm, *scratch)

    kern = pl.kernel(
        body,
        mesh=mesh,
        compiler_params=pltpu.CompilerParams(use_tc_tiling_on_sc=False),
        out_type=jax.ShapeDtypeStruct((nchunks, N_NODE, HALF), jnp.float32),
        scratch_types=[
            pltpu.VMEM_SHARED((N_NODE, HALF), jnp.float32),
            pltpu.VMEM((ESTAGE,), jnp.int32),
            pltpu.VMEM((ESTAGE,), jnp.int32),
            pltpu.VMEM((ESTAGE,), jnp.float32),
            pltpu.VMEM((EDGE_CHUNK, HALF), jnp.float32),
            pltpu.VMEM((EDGE_CHUNK, HALF), jnp.float32),
            pltpu.VMEM((ZCHUNK, HALF), jnp.float32),
            pltpu.SemaphoreType.DMA,
            pltpu.SemaphoreType.DMA,
            pltpu.SemaphoreType.DMA,
            pltpu.SemaphoreType.DMA,
        ],
    )
    return kern(xflat, cols_k, rows_k, vals_k)


GIDX_CHUNK = 80


def _gather_sc(table, idx):
    n = idx.shape[0]
    per_w = n // (NC * NS)
    mesh = plsc.VectorSubcoreMesh(core_axis_name="c", subcore_axis_name="s")

    def body(table_hbm, idx_hbm, out_hbm, idx_v, rows_v, sem):
        wid = lax.axis_index("s") * NC + lax.axis_index("c")
        base = wid * per_w

        def step(i, c):
            off = base + i * GIDX_CHUNK
            pltpu.sync_copy(idx_hbm.at[pl.ds(off, GIDX_CHUNK)], idx_v)
            pltpu.async_copy(table_hbm.at[idx_v], rows_v, sem).wait()
            pltpu.sync_copy(rows_v, out_hbm.at[pl.ds(off, GIDX_CHUNK)])
            return c

        lax.fori_loop(0, per_w // GIDX_CHUNK, step, 0)

    kern = pl.kernel(
        body,
        mesh=mesh,
        out_type=jax.ShapeDtypeStruct((n, EMB), jnp.float32),
        scratch_types=[
            pltpu.VMEM((GIDX_CHUNK,), jnp.int32),
            pltpu.VMEM((GIDX_CHUNK, EMB), jnp.float32),
            pltpu.SemaphoreType.DMA,
        ],
    )
    return kern(table, idx)


def _mm_kernel(x_ref, w_ref, o_ref):
    o_ref[...] = jax.lax.dot_general(
        x_ref[...], w_ref[...], (((1,), (0,)), ((), ())),
        preferred_element_type=jnp.float32)


def _matmul(x, w, block_m=2048):
    m, k = x.shape
    n = w.shape[1]
    nblk = pl.cdiv(m, block_m)
    return pl.pallas_call(
        _mm_kernel,
        grid=(nblk,),
        in_specs=[pl.BlockSpec((block_m, k), lambda i: (i, 0)),
                  pl.BlockSpec((k, n), lambda i: (0, 0))],
        out_specs=pl.BlockSpec((block_m, n), lambda i: (i, 0)),
        out_shape=jax.ShapeDtypeStruct((m, n), jnp.float32),
    )(x, w)


def _l2n(x, eps=1e-12):
    n = jnp.sqrt(jnp.sum(x * x, axis=-1, keepdims=True))
    return x / jnp.maximum(n, eps)


def _combine_kernel(x0_ref, z1_ref, z2_ref, mixin_ref, itemn_ref):
    x0 = x0_ref[...]
    z1 = z1_ref[...]
    z2 = z2_ref[...]
    outs = []
    for b in range(5):
        sl = slice(b * EMB, (b + 1) * EMB)
        outs.append((x0[:, sl] + _l2n(z1[:, sl]) + _l2n(z2[:, sl])) / 3.0)
    item_emb = outs[0]
    image_emb = outs[1] + outs[3]
    text_emb = outs[2] + outs[4]
    mixin_ref[...] = jnp.concatenate([item_emb, image_emb, text_emb], axis=-1)
    itemn_ref[...] = _l2n(item_emb)


def _combine(x0cat, z1cat, z2cat, block_m=2000):
    nblk = N_NODE // block_m
    return pl.pallas_call(
        _combine_kernel,
        grid=(nblk,),
        in_specs=[pl.BlockSpec((block_m, 5 * EMB), lambda i: (i, 0))] * 3,
        out_specs=[pl.BlockSpec((block_m, 3 * EMB), lambda i: (i, 0)),
                   pl.BlockSpec((block_m, EMB), lambda i: (i, 0))],
        out_shape=[jax.ShapeDtypeStruct((N_NODE, 3 * EMB), jnp.float32),
                   jax.ShapeDtypeStruct((N_NODE, EMB), jnp.float32)],
    )(x0cat, z1cat, z2cat)


def _mlp_kernel(x_ref, w1_ref, b1_ref, w2_ref, b2_ref, o_ref):
    h = jnp.tanh(jax.lax.dot_general(
        x_ref[...], w1_ref[...], (((1,), (0,)), ((), ())),
        preferred_element_type=jnp.float32) + b1_ref[...])
    o_ref[...] = jnp.tanh(jax.lax.dot_general(
        h, w2_ref[...], (((1,), (0,)), ((), ())),
        preferred_element_type=jnp.float32) + b2_ref[...])


def _mlp(x, w1t, b1, w2t, b2, block_m=2000):
    nblk = N_NODE // block_m
    return pl.pallas_call(
        _mlp_kernel,
        grid=(nblk,),
        in_specs=[pl.BlockSpec((block_m, 3 * EMB), lambda i: (i, 0)),
                  pl.BlockSpec((3 * EMB, EMB), lambda i: (0, 0)),
                  pl.BlockSpec((1, EMB), lambda i: (0, 0)),
                  pl.BlockSpec((EMB, EMB), lambda i: (0, 0)),
                  pl.BlockSpec((1, EMB), lambda i: (0, 0))],
        out_specs=pl.BlockSpec((block_m, EMB), lambda i: (i, 0)),
        out_shape=jax.ShapeDtypeStruct((N_NODE, EMB), jnp.float32),
    )(x, w1t, b1.reshape(1, EMB), w2t, b2.reshape(1, EMB))


SESS_BLK = 128


def _attn_kernel(seqh_ref, len_ref, mask_ref, g1w_ref, g1b_ref, g2w_ref,
                 w2_ref, sess_ref):
    seq_h = seqh_ref[...].reshape(SESS_BLK, L, EMB)
    lens = len_ref[...].astype(jnp.float32)
    maskf = mask_ref[...][..., None]
    hs = jnp.sum(seq_h, axis=1) / lens
    flat = seq_h.reshape(SESS_BLK * L, EMB)
    nh_lin = jax.lax.dot_general(jnp.tanh(flat), g1w_ref[...],
                                 (((1,), (0,)), ((), ())),
                                 preferred_element_type=jnp.float32)
    hs_lin = jax.lax.dot_general(hs, g2w_ref[...], (((1,), (0,)), ((), ())),
                                 preferred_element_type=jnp.float32)
    nh = jax.nn.sigmoid(nh_lin.reshape(SESS_BLK, L, EMB) + g1b_ref[...]
                        + hs_lin[:, None, :])
    beta = jax.lax.dot_general(nh.reshape(SESS_BLK * L, EMB), w2_ref[...],
                               (((1,), (0,)), ((), ())),
                               preferred_element_type=jnp.float32)
    beta = beta.reshape(SESS_BLK, L, 1) * maskf
    select = jnp.sum(beta * seq_h, axis=1)

    pos = lax.broadcasted_iota(jnp.int32, (SESS_BLK, L), 1).astype(jnp.float32)
    lens_b = lens
    order = jnp.where(pos < lens_b, lens_b - pos, 0.0)
    new_order = jnp.exp(order / T2)
    last = seq_h[:, 0:1, :]
    dot = jnp.sum(seq_h * last, axis=-1)
    na = jnp.sqrt(jnp.sum(seq_h * seq_h, axis=-1))
    nb = jnp.sqrt(jnp.sum(last * last, axis=-1))
    cs = dot / (jnp.maximum(na, 1e-8) * jnp.maximum(nb, 1e-8))
    weights = new_order * cs
    wmask = jnp.where(weights != 0, weights, -9e10)
    wmax = jnp.max(wmask, axis=1, keepdims=True)
    ew = jnp.exp(wmask - wmax)
    fw = ew / jnp.sum(ew, axis=1, keepdims=True)
    session_aw = jnp.sum(fw[..., None] * seq_h, axis=1)
    sess_ref[...] = select + session_aw


def _attention(seq_h_flat, session_len, mask, g1w_t, g1b, g2w_t, w_2):
    nblk = B // SESS_BLK
    return pl.pallas_call(
        _attn_kernel,
        grid=(nblk,),
        in_specs=[pl.BlockSpec((SESS_BLK * L, EMB), lambda i: (i, 0)),
                  pl.BlockSpec((SESS_BLK, 1), lambda i: (i, 0)),
                  pl.BlockSpec((SESS_BLK, L), lambda i: (i, 0)),
                  pl.BlockSpec((EMB, EMB), lambda i: (0, 0)),
                  pl.BlockSpec((1, EMB), lambda i: (0, 0)),
                  pl.BlockSpec((EMB, EMB), lambda i: (0, 0)),
                  pl.BlockSpec((EMB, 1), lambda i: (0, 0))],
        out_specs=pl.BlockSpec((SESS_BLK, EMB), lambda i: (i, 0)),
        out_shape=jax.ShapeDtypeStruct((B, EMB), jnp.float32),
    )(seq_h_flat, session_len, mask, g1w_t, g1b.reshape(1, EMB), g2w_t, w_2)


def _simtop_kernel(sess_ref, out_ref):
    s = sess_ref[...]
    fenzi = jax.lax.dot_general(s, s, (((1,), (1,)), ((), ())),
                                preferred_element_type=jnp.float32)
    fenmu_l = jnp.sqrt(jnp.sum(s * s + 1e-6, axis=1, keepdims=True))
    denom = fenmu_l * fenmu_l.reshape(1, B)
    logits = fenzi / denom
    lmax = jnp.max(logits, axis=1, keepdims=True)
    el = jnp.exp(logits - lmax)
    cos_sim = el / jnp.sum(el, axis=1, keepdims=True)

    cols = lax.broadcasted_iota(jnp.int32, (B, B), 1)
    work = cos_sim
    vals = []
    hots = []
    for _ in range(3):
        v = jnp.max(work, axis=1, keepdims=True)
        ismax = work == v
        first = jnp.min(jnp.where(ismax, cols, B), axis=1, keepdims=True)
        hot = (cols == first).astype(jnp.float32)
        vals.append(v)
        hots.append(hot)
        work = jnp.where(hot > 0, -jnp.inf, work)
    v3 = jnp.concatenate(vals, axis=1)
    vmax = jnp.max(v3, axis=1, keepdims=True)
    ev = jnp.exp(v3 - vmax)
    wsm = ev / jnp.sum(ev, axis=1, keepdims=True)
    m = (wsm[:, 0:1] * hots[0] + wsm[:, 1:2] * hots[1]
         + wsm[:, 2:3] * hots[2])
    neighbor = jax.lax.dot_general(m, s, (((1,), (0,)), ((), ())),
                                   preferred_element_type=jnp.float32)
    sess_final = s + _l2n(neighbor)
    out_ref[...] = W_K * _l2n(sess_final)


def _simtop(sess_emb):
    return pl.pallas_call(
        _simtop_kernel,
        out_shape=jax.ShapeDtypeStruct((B, EMB), jnp.float32),
    )(sess_emb)


def _scores_kernel(sess_ref, itemn_ref, o_ref):
    o_ref[...] = jax.lax.dot_general(
        sess_ref[...], itemn_ref[...], (((1,), (1,)), ((), ())),
        preferred_element_type=jnp.float32)


def _scores(sess, item_n, block_n=4096):
    nblk = pl.cdiv(N_NODE, block_n)
    return pl.pallas_call(
        _scores_kernel,
        grid=(nblk,),
        in_specs=[pl.BlockSpec((B, EMB), lambda i: (0, 0)),
                  pl.BlockSpec((block_n, EMB), lambda i: (i, 0))],
        out_specs=pl.BlockSpec((B, block_n), lambda i: (0, i)),
        out_shape=jax.ShapeDtypeStruct((B, N_NODE), jnp.float32),
    )(sess, item_n)


def kernel(session_item, session_len, reversed_sess_item, mask, embedding,
           image_pca, text_pca, adj_rows, adj_cols, adj_vals, img_rows,
           img_cols, img_vals, txt_rows, txt_cols, txt_vals, W_ic0, W_ic1,
           glu1_w, glu1_b, glu2_w, w_2, mlp1_w, mlp1_b, mlp2_w, mlp2_b):
    rows_list = [adj_rows, img_rows, txt_rows]
    cols_list = [adj_cols, img_cols, txt_cols]
    vals_list = [adj_vals, img_vals, txt_vals]

    x_idx1 = [0, 1, 2, 3, 4, 5, 2, 3, 4, 5]
    graphs = [0, 0, 0, 0, 0, 0, 1, 1, 2, 2]
    rows_k = jnp.stack([rows_list[g] for g in graphs]).astype(jnp.int32)
    vals_k = jnp.stack([vals_list[g] for g in graphs])
    cols_g = [c.astype(jnp.int32) for c in cols_list]
    cols_k1 = jnp.stack([cols_g[graphs[kk]] + x_idx1[kk] * N_NODE
                         for kk in range(10)])
    cols_k2 = jnp.stack([cols_g[graphs[kk]] + kk * N_NODE
                         for kk in range(10)])

    x3 = jnp.concatenate([embedding, image_pca, text_pca], axis=0)
    y1 = _matmul(x3, W_ic0.T)
    y1c = y1.reshape(3, N_NODE, 2, HALF).transpose(0, 2, 1, 3)
    z1 = _spmm_sc(y1c.reshape(6 * N_NODE, HALF), cols_k1, rows_k,
                  vals_k)
    z1b = z1.reshape(5, 2, N_NODE, HALF).transpose(0, 2, 1, 3)
    z1b = z1b.reshape(5, N_NODE, EMB)

    y2 = _matmul(z1b.reshape(5 * N_NODE, EMB), W_ic1.T)
    y2c = y2.reshape(5, N_NODE, 2, HALF).transpose(0, 2, 1, 3)
    z2 = _spmm_sc(y2c.reshape(10 * N_NODE, HALF), cols_k2, rows_k, vals_k)
    z2b = z2.reshape(5, 2, N_NODE, HALF).transpose(0, 2, 1, 3)
    z2b = z2b.reshape(5, N_NODE, EMB)

    x0cat = jnp.concatenate([embedding, image_pca, text_pca, image_pca,
                             text_pca], axis=1)
    z1cat = jnp.concatenate([z1b[i] for i in range(5)], axis=1)
    z2cat = jnp.concatenate([z2b[i] for i in range(5)], axis=1)
    mixin, item_n = _combine(x0cat, z1cat, z2cat)

    mix = _mlp(mixin, mlp1_w.T, mlp1_b, mlp2_w.T, mlp2_b)

    table = jnp.concatenate([jnp.zeros((1, EMB), jnp.float32), mix], axis=0)
    seq_h_flat = _gather_sc(table, reversed_sess_item.reshape(B * L)
                            .astype(jnp.int32))
    sess_emb = _attention(seq_h_flat, session_len, mask, glu1_w.T, glu1_b,
                          glu2_w.T, w_2)
    sess = _simtop(sess_emb)
    return _scores(sess, item_n)


# speedup vs baseline: 2.0760x; 1.1211x over previous
import functools

import jax
import jax.numpy as jnp
from jax import lax
from jax.experimental import pallas as pl
from jax.experimental.pallas import tpu as pltpu
from jax.experimental.pallas import tpu_sc as plsc

N_NODE = 20000
EMB = 128
B = 1024
L = 50
E = 320000
T2 = 10.0
W_K = 10.0

NC = 2
NS = 16
HALF = 64


EDGE_CHUNK = 80
EDGES_PER_SUB = E // NS
SLAB = 1000
NSLAB = N_NODE // SLAB
ZCHUNK = 40
ESTAGE = 10000


def _spmm_body(nchunks, xref, ecols, erows, evals, out_hbm,
               acc_shared, idx_v, row_v, val_v, gbuf, gbuf2, zbuf,
               sem, sem2, sem3, sem4):
    cid = lax.axis_index("c")
    sid = lax.axis_index("s")

    def zrow(i, c):
        for j in range(HALF // 16):
            zbuf[i, pl.ds(j * 16, 16)] = jnp.zeros((16,), jnp.float32)
        return c

    lax.fori_loop(0, ZCHUNK, zrow, 0)

    def do_chunk(i_chunk, carry):
        k = i_chunk * NC + cid
        for rep in range(NSLAB // NS + 1):
            slab = rep * NS + sid

            @pl.when(slab < NSLAB)
            def _():
                def zs(z, c):
                    pltpu.sync_copy(
                        zbuf,
                        acc_shared.at[pl.ds(slab * SLAB + z * ZCHUNK, ZCHUNK)])
                    return c

                lax.fori_loop(0, SLAB // ZCHUNK, zs, 0)
        plsc.subcore_barrier()

        gb = (gbuf, gbuf2)
        gsem = (sem, sem2)
        ssem = (sem3, sem4)
        nstep = ESTAGE // EDGE_CHUNK

        def gather(i, p):
            pltpu.async_copy(
                xref.at[idx_v.at[pl.ds(i * EDGE_CHUNK, EDGE_CHUNK)]],
                gb[p], gsem[p])

        def scale(i, p):
            buf = gb[p]

            def grp_body(grp, c2):
                vv = val_v[pl.ds(i * EDGE_CHUNK + grp * 16, 16)]
                for l in range(16):
                    s = vv[l]
                    e = grp * 16 + l
                    for j in range(HALF // 16):
                        sl = pl.ds(j * 16, 16)
                        buf[e, sl] = buf[e, sl] * s
                return c2

            lax.fori_loop(0, EDGE_CHUNK // 16, grp_body, 0)

        def scatter(i, p):
            for q in range(EDGE_CHUNK // 16):
                idx16 = row_v[pl.ds(i * EDGE_CHUNK + q * 16, 16)]
                pltpu.async_copy(gb[p].at[pl.ds(q * 16, 16)],
                                 acc_shared.at[idx16], ssem[p], add=True)

        def gwait(i, p):
            pltpu.make_async_copy(
                xref.at[idx_v.at[pl.ds(i * EDGE_CHUNK, EDGE_CHUNK)]],
                gb[p], gsem[p]).wait()

        def swait(i, p):
            for q in range(EDGE_CHUNK // 16):
                idx16 = row_v[pl.ds(i * EDGE_CHUNK + q * 16, 16)]
                pltpu.make_async_copy(gb[p].at[pl.ds(q * 16, 16)],
                                      acc_shared.at[idx16], ssem[p]).wait()

        def pair(j, carry):
            e = 2 * j
            gwait(e, 0)
            gather(e + 1, 1)
            scale(e, 0)
            scatter(e, 0)
            swait(e, 0)
            gwait(e + 1, 1)
            gather(e + 2, 0)
            scale(e + 1, 1)
            scatter(e + 1, 1)
            swait(e + 1, 1)
            return carry

        def half(h, carry):
            ebase = sid * EDGES_PER_SUB + h * ESTAGE
            pltpu.sync_copy(ecols.at[k, pl.ds(ebase, ESTAGE)], idx_v)
            pltpu.sync_copy(erows.at[k, pl.ds(ebase, ESTAGE)], row_v)
            pltpu.sync_copy(evals.at[k, pl.ds(ebase, ESTAGE)], val_v)
            gather(0, 0)
            lax.fori_loop(0, (nstep - 1) // 2, pair, 0)
            e = nstep - 1
            gwait(e, 0)
            scale(e, 0)
            scatter(e, 0)
            swait(e, 0)
            return carry

        lax.fori_loop(0, EDGES_PER_SUB // ESTAGE, half, 0)
        plsc.subcore_barrier()
        for rep in range(NSLAB // NS + 1):
            slab = rep * NS + sid

            @pl.when(slab < NSLAB)
            def _():
                pltpu.sync_copy(acc_shared.at[pl.ds(slab * SLAB, SLAB)],
                                out_hbm.at[k, pl.ds(slab * SLAB, SLAB)])
        plsc.subcore_barrier()
        return carry

    lax.fori_loop(0, nchunks // NC, do_chunk, 0)


def _spmm_sc(xflat, cols_k, rows_k, vals_k):
    nchunks = cols_k.shape[0]
    mesh = plsc.VectorSubcoreMesh(core_axis_name="c", subcore_axis_name="s")

    def body(xref, ecols, erows, evals, out_hbm, *scratch):
        _spmm_body(nchunks, xref, ecols, erows, evals, out_hbm, *scratch)

    kern = pl.kernel(
        body,
        mesh=mesh,
        compiler_params=pltpu.CompilerParams(use_tc_tiling_on_sc=False),
        out_type=jax.ShapeDtypeStruct((nchunks, N_NODE, HALF), jnp.float32),
        scratch_types=[
            pltpu.VMEM_SHARED((N_NODE, HALF), jnp.float32),
            pltpu.VMEM((ESTAGE,), jnp.int32),
            pltpu.VMEM((ESTAGE,), jnp.int32),
            pltpu.VMEM((ESTAGE,), jnp.float32),
            pltpu.VMEM((EDGE_CHUNK, HALF), jnp.float32),
            pltpu.VMEM((EDGE_CHUNK, HALF), jnp.float32),
            pltpu.VMEM((ZCHUNK, HALF), jnp.float32),
            pltpu.SemaphoreType.DMA,
            pltpu.SemaphoreType.DMA,
            pltpu.SemaphoreType.DMA,
            pltpu.SemaphoreType.DMA,
        ],
    )
    return kern(xflat, cols_k, rows_k, vals_k)


GIDX_CHUNK = 80


def _gather_sc(table, idx):
    n = idx.shape[0]
    per_w = n // (NC * NS)
    mesh = plsc.VectorSubcoreMesh(core_axis_name="c", subcore_axis_name="s")

    def body(table_hbm, idx_hbm, out_hbm, idx_v, rows_v, sem):
        wid = lax.axis_index("s") * NC + lax.axis_index("c")
        base = wid * per_w

        def step(i, c):
            off = base + i * GIDX_CHUNK
            pltpu.sync_copy(idx_hbm.at[pl.ds(off, GIDX_CHUNK)], idx_v)
            pltpu.async_copy(table_hbm.at[idx_v], rows_v, sem).wait()
            pltpu.sync_copy(rows_v, out_hbm.at[pl.ds(off, GIDX_CHUNK)])
            return c

        lax.fori_loop(0, per_w // GIDX_CHUNK, step, 0)

    kern = pl.kernel(
        body,
        mesh=mesh,
        out_type=jax.ShapeDtypeStruct((n, EMB), jnp.float32),
        scratch_types=[
            pltpu.VMEM((GIDX_CHUNK,), jnp.int32),
            pltpu.VMEM((GIDX_CHUNK, EMB), jnp.float32),
            pltpu.SemaphoreType.DMA,
        ],
    )
    return kern(table, idx)


def _mm_kernel(x_ref, w_ref, o_ref):
    o_ref[...] = jax.lax.dot_general(
        x_ref[...], w_ref[...], (((1,), (0,)), ((), ())),
        preferred_element_type=jnp.float32)


def _matmul(x, w, block_m=2048):
    m, k = x.shape
    n = w.shape[1]
    nblk = pl.cdiv(m, block_m)
    return pl.pallas_call(
        _mm_kernel,
        grid=(nblk,),
        in_specs=[pl.BlockSpec((block_m, k), lambda i: (i, 0)),
                  pl.BlockSpec((k, n), lambda i: (0, 0))],
        out_specs=pl.BlockSpec((block_m, n), lambda i: (i, 0)),
        out_shape=jax.ShapeDtypeStruct((m, n), jnp.float32),
    )(x, w)


def _l2n(x, eps=1e-12):
    n = jnp.sqrt(jnp.sum(x * x, axis=-1, keepdims=True))
    return x / jnp.maximum(n, eps)


def _combine_kernel(emb_ref, img_ref, txt_ref, z1_ref, z2_ref, mixin_ref,
                    itemn_ref):
    x0 = [emb_ref[...], img_ref[...], txt_ref[...]]
    x0 = [x0[0], x0[1], x0[2], x0[1], x0[2]]
    z1 = z1_ref[...]
    z2 = z2_ref[...]
    outs = []
    for b in range(5):
        z2b = jnp.concatenate([z2[2 * b], z2[2 * b + 1]], axis=-1)
        outs.append((x0[b] + _l2n(z1[b]) + _l2n(z2b)) / 3.0)
    item_emb = outs[0]
    image_emb = outs[1] + outs[3]
    text_emb = outs[2] + outs[4]
    mixin_ref[...] = jnp.concatenate([item_emb, image_emb, text_emb], axis=-1)
    itemn_ref[...] = _l2n(item_emb)


def _combine(emb, img, txt, z1b, z2, block_m=2000):
    nblk = N_NODE // block_m
    return pl.pallas_call(
        _combine_kernel,
        grid=(nblk,),
        in_specs=[pl.BlockSpec((block_m, EMB), lambda i: (i, 0)),
                  pl.BlockSpec((block_m, EMB), lambda i: (i, 0)),
                  pl.BlockSpec((block_m, EMB), lambda i: (i, 0)),
                  pl.BlockSpec((5, block_m, EMB), lambda i: (0, i, 0)),
                  pl.BlockSpec((10, block_m, HALF), lambda i: (0, i, 0))],
        out_specs=[pl.BlockSpec((block_m, 3 * EMB), lambda i: (i, 0)),
                   pl.BlockSpec((block_m, EMB), lambda i: (i, 0))],
        out_shape=[jax.ShapeDtypeStruct((N_NODE, 3 * EMB), jnp.float32),
                   jax.ShapeDtypeStruct((N_NODE, EMB), jnp.float32)],
    )(emb, img, txt, z1b, z2)


def _mlp_kernel(x_ref, w1_ref, b1_ref, w2_ref, b2_ref, o_ref):
    h = jnp.tanh(jax.lax.dot_general(
        x_ref[...], w1_ref[...], (((1,), (0,)), ((), ())),
        preferred_element_type=jnp.float32) + b1_ref[...])
    o_ref[...] = jnp.tanh(jax.lax.dot_general(
        h, w2_ref[...], (((1,), (0,)), ((), ())),
        preferred_element_type=jnp.float32) + b2_ref[...])


def _mlp(x, w1t, b1, w2t, b2, block_m=2000):
    nblk = N_NODE // block_m
    return pl.pallas_call(
        _mlp_kernel,
        grid=(nblk,),
        in_specs=[pl.BlockSpec((block_m, 3 * EMB), lambda i: (i, 0)),
                  pl.BlockSpec((3 * EMB, EMB), lambda i: (0, 0)),
                  pl.BlockSpec((1, EMB), lambda i: (0, 0)),
                  pl.BlockSpec((EMB, EMB), lambda i: (0, 0)),
                  pl.BlockSpec((1, EMB), lambda i: (0, 0))],
        out_specs=pl.BlockSpec((block_m, EMB), lambda i: (i, 0)),
        out_shape=jax.ShapeDtypeStruct((N_NODE, EMB), jnp.float32),
    )(x, w1t, b1.reshape(1, EMB), w2t, b2.reshape(1, EMB))


SESS_BLK = 128


def _attn_kernel(seqh_ref, len_ref, mask_ref, g1w_ref, g1b_ref, g2w_ref,
                 w2_ref, sess_ref):
    seq_h = seqh_ref[...].reshape(SESS_BLK, L, EMB)
    lens = len_ref[...].astype(jnp.float32)
    maskf = mask_ref[...][..., None]
    hs = jnp.sum(seq_h, axis=1) / lens
    flat = seq_h.reshape(SESS_BLK * L, EMB)
    nh_lin = jax.lax.dot_general(jnp.tanh(flat), g1w_ref[...],
                                 (((1,), (0,)), ((), ())),
                                 preferred_element_type=jnp.float32)
    hs_lin = jax.lax.dot_general(hs, g2w_ref[...], (((1,), (0,)), ((), ())),
                                 preferred_element_type=jnp.float32)
    nh = jax.nn.sigmoid(nh_lin.reshape(SESS_BLK, L, EMB) + g1b_ref[...]
                        + hs_lin[:, None, :])
    beta = jax.lax.dot_general(nh.reshape(SESS_BLK * L, EMB), w2_ref[...],
                               (((1,), (0,)), ((), ())),
                               preferred_element_type=jnp.float32)
    beta = beta.reshape(SESS_BLK, L, 1) * maskf
    select = jnp.sum(beta * seq_h, axis=1)

    pos = lax.broadcasted_iota(jnp.int32, (SESS_BLK, L), 1).astype(jnp.float32)
    lens_b = lens
    order = jnp.where(pos < lens_b, lens_b - pos, 0.0)
    new_order = jnp.exp(order / T2)
    last = seq_h[:, 0:1, :]
    dot = jnp.sum(seq_h * last, axis=-1)
    na = jnp.sqrt(jnp.sum(seq_h * seq_h, axis=-1))
    nb = jnp.sqrt(jnp.sum(last * last, axis=-1))
    cs = dot / (jnp.maximum(na, 1e-8) * jnp.maximum(nb, 1e-8))
    weights = new_order * cs
    wmask = jnp.where(weights != 0, weights, -9e10)
    wmax = jnp.max(wmask, axis=1, keepdims=True)
    ew = jnp.exp(wmask - wmax)
    fw = ew / jnp.sum(ew, axis=1, keepdims=True)
    session_aw = jnp.sum(fw[..., None] * seq_h, axis=1)
    sess_ref[...] = select + session_aw


def _attention(seq_h_flat, session_len, mask, g1w_t, g1b, g2w_t, w_2):
    nblk = B // SESS_BLK
    return pl.pallas_call(
        _attn_kernel,
        grid=(nblk,),
        in_specs=[pl.BlockSpec((SESS_BLK * L, EMB), lambda i: (i, 0)),
                  pl.BlockSpec((SESS_BLK, 1), lambda i: (i, 0)),
                  pl.BlockSpec((SESS_BLK, L), lambda i: (i, 0)),
                  pl.BlockSpec((EMB, EMB), lambda i: (0, 0)),
                  pl.BlockSpec((1, EMB), lambda i: (0, 0)),
                  pl.BlockSpec((EMB, EMB), lambda i: (0, 0)),
                  pl.BlockSpec((EMB, 1), lambda i: (0, 0))],
        out_specs=pl.BlockSpec((SESS_BLK, EMB), lambda i: (i, 0)),
        out_shape=jax.ShapeDtypeStruct((B, EMB), jnp.float32),
    )(seq_h_flat, session_len, mask, g1w_t, g1b.reshape(1, EMB), g2w_t, w_2)


def _simtop_kernel(sess_ref, out_ref):
    s = sess_ref[...]
    fenzi = jax.lax.dot_general(s, s, (((1,), (1,)), ((), ())),
                                preferred_element_type=jnp.float32)
    fenmu_l = jnp.sqrt(jnp.sum(s * s + 1e-6, axis=1, keepdims=True))
    denom = fenmu_l * fenmu_l.reshape(1, B)
    logits = fenzi / denom
    lmax = jnp.max(logits, axis=1, keepdims=True)
    el = jnp.exp(logits - lmax)
    cos_sim = el / jnp.sum(el, axis=1, keepdims=True)

    cols = lax.broadcasted_iota(jnp.int32, (B, B), 1)
    work = cos_sim
    vals = []
    hots = []
    for _ in range(3):
        v = jnp.max(work, axis=1, keepdims=True)
        ismax = work == v
        first = jnp.min(jnp.where(ismax, cols, B), axis=1, keepdims=True)
        hot = (cols == first).astype(jnp.float32)
        vals.append(v)
        hots.append(hot)
        work = jnp.where(hot > 0, -jnp.inf, work)
    v3 = jnp.concatenate(vals, axis=1)
    vmax = jnp.max(v3, axis=1, keepdims=True)
    ev = jnp.exp(v3 - vmax)
    wsm = ev / jnp.sum(ev, axis=1, keepdims=True)
    m = (wsm[:, 0:1] * hots[0] + wsm[:, 1:2] * hots[1]
         + wsm[:, 2:3] * hots[2])
    neighbor = jax.lax.dot_general(m, s, (((1,), (0,)), ((), ())),
                                   preferred_element_type=jnp.float32)
    sess_final = s + _l2n(neighbor)
    out_ref[...] = W_K * _l2n(sess_final)


def _simtop(sess_emb):
    return pl.pallas_call(
        _simtop_kernel,
        out_shape=jax.ShapeDtypeStruct((B, EMB), jnp.float32),
    )(sess_emb)


def _scores_kernel(sess_ref, itemn_ref, o_ref):
    o_ref[...] = jax.lax.dot_general(
        sess_ref[...], itemn_ref[...], (((1,), (1,)), ((), ())),
        preferred_element_type=jnp.float32)


def _scores(sess, item_n, block_n=4096):
    nblk = pl.cdiv(N_NODE, block_n)
    return pl.pallas_call(
        _scores_kernel,
        grid=(nblk,),
        in_specs=[pl.BlockSpec((B, EMB), lambda i: (0, 0)),
                  pl.BlockSpec((block_n, EMB), lambda i: (i, 0))],
        out_specs=pl.BlockSpec((B, block_n), lambda i: (0, i)),
        out_shape=jax.ShapeDtypeStruct((B, N_NODE), jnp.float32),
    )(sess, item_n)


def kernel(session_item, session_len, reversed_sess_item, mask, embedding,
           image_pca, text_pca, adj_rows, adj_cols, adj_vals, img_rows,
           img_cols, img_vals, txt_rows, txt_cols, txt_vals, W_ic0, W_ic1,
           glu1_w, glu1_b, glu2_w, w_2, mlp1_w, mlp1_b, mlp2_w, mlp2_b):
    rows_list = [adj_rows, img_rows, txt_rows]
    cols_list = [adj_cols, img_cols, txt_cols]
    vals_list = [adj_vals, img_vals, txt_vals]

    x_idx1 = [0, 1, 2, 3, 4, 5, 2, 3, 4, 5]
    graphs = [0, 0, 0, 0, 0, 0, 1, 1, 2, 2]
    rows_k = jnp.stack([rows_list[g] for g in graphs]).astype(jnp.int32)
    vals_k = jnp.stack([vals_list[g] for g in graphs])
    cols_g = [c.astype(jnp.int32) for c in cols_list]
    cols_k1 = jnp.stack([2 * (cols_g[graphs[kk]] + (x_idx1[kk] // 2) * N_NODE)
                         + (kk % 2) for kk in range(10)])
    cols_k2 = jnp.stack([2 * (cols_g[graphs[kk]] + (kk // 2) * N_NODE)
                         + (kk % 2) for kk in range(10)])

    x3 = jnp.concatenate([embedding, image_pca, text_pca], axis=0)
    y1 = _matmul(x3, W_ic0.T)
    z1 = _spmm_sc(y1.reshape(6 * N_NODE, HALF), cols_k1, rows_k,
                  vals_k)
    z1b = z1.reshape(5, 2, N_NODE, HALF).transpose(0, 2, 1, 3)
    z1b = z1b.reshape(5, N_NODE, EMB)

    y2 = _matmul(z1b.reshape(5 * N_NODE, EMB), W_ic1.T)
    z2 = _spmm_sc(y2.reshape(10 * N_NODE, HALF), cols_k2, rows_k, vals_k)

    mixin, item_n = _combine(embedding, image_pca, text_pca, z1b, z2)

    mix = _mlp(mixin, mlp1_w.T, mlp1_b, mlp2_w.T, mlp2_b)

    table = jnp.concatenate([jnp.zeros((1, EMB), jnp.float32), mix], axis=0)
    seq_h_flat = _gather_sc(table, reversed_sess_item.reshape(B * L)
                            .astype(jnp.int32))
    sess_emb = _attention(seq_h_flat, session_len, mask, glu1_w.T, glu1_b,
                          glu2_w.T, w_2)
    sess = _simtop(sess_emb)
    return _scores(sess, item_n)
